# baseline (speedup 1.0000x reference)
import functools

import numpy as np
import jax
import jax.numpy as jnp
from jax import lax
from jax.experimental import pallas as pl
from jax.experimental.pallas import tpu as pltpu
from jax.experimental.pallas import tpu_sc as plsc

_N = 200
_NO = _N - 1
_TS_PAD = 208
_POS_PAD = 512
_TSW_PAD = 256
_OUT_WORDS = _NO * _NO
_OUT_STRIDE = 39680
_OUT_BUF = 39680
_LUT_SIZE = 588
_LUT_PAD = 640


def _build_lut() -> np.ndarray:
    d = np.arange(1, 1 << 20, dtype=np.int64)
    f = d.astype(np.float32)
    key = (f.view(np.int32) >> 21).astype(np.int64)
    b = (np.log(f) / np.float32(0.301)).astype(np.int32)
    minb = np.full(_LUT_SIZE, 127, np.int32)
    maxb = np.full(_LUT_SIZE, -1, np.int32)
    np.minimum.at(minb, key, b)
    np.maximum.at(maxb, key, b)
    used = maxb >= 0
    assert np.all((maxb - minb)[used] <= 1), "chunk crosses >1 bucket boundary"
    thr = np.full(_LUT_SIZE, 0xFFFFFF, np.int64)
    grew = b > minb[key]
    np.minimum.at(thr, key[grew], d[grew])
    base = np.where(used, minb, 0).astype(np.int64)
    packed = (base << 24) | thr
    return packed.astype(np.int32)


_LUT = _build_lut()
_LUT_REP = np.repeat(_LUT, 16)


def _sc_body(bpw, nc, ts_hbm, tsw_hbm, pos_hbm, lut_hbm, out_hbm,
             ts_all_v, tsw_v, pos_v, lut_v, out_v0, out_v1, sem0, sem1):
    out_bufs = (out_v0, out_v1)
    wid = lax.axis_index("s") * nc + lax.axis_index("c")
    b0 = wid * bpw
    _sc_work(bpw, b0, ts_hbm, tsw_hbm, pos_hbm, lut_hbm, out_hbm,
             ts_all_v, tsw_v, pos_v, lut_v, out_bufs, (sem0, sem1))


def _sc_work(bpw, b0, ts_hbm, tsw_hbm, pos_hbm, lut_hbm, out_hbm,
             ts_all_v, tsw_v, pos_v, lut_v, out_bufs, sems):
    pltpu.sync_copy(tsw_hbm, tsw_v)
    pltpu.sync_copy(pos_hbm, pos_v)
    pltpu.sync_copy(lut_hbm, lut_v)
    pltpu.sync_copy(ts_hbm.at[pl.ds(b0 * _TS_PAD, bpw * _TS_PAD)], ts_all_v)

    zeros16 = jnp.zeros((16,), jnp.int32)
    ones16 = jnp.full((16,), 1, jnp.int32)
    lane = lax.iota(jnp.int32, 16)
    tailcols = jnp.minimum(lane + 192, _NO - 1)
    tailmask = lane < (_NO - 192)

    nj = 13

    def compute_b(k, buf):
        ts_base = k * _TS_PAD
        tsj = [ts_all_v[pl.ds(ts_base + jv * 16, 16)] for jv in range(nj)]

        def row(i, carry):
            ti = plsc.load_gather(ts_all_v, [zeros16 + (ts_base + i + 1)])
            poff = _NO - i
            d = [jnp.abs(ti - tsj[jv]) for jv in range(nj)]
            key = [lax.shift_right_logical(
                plsc.bitcast(d[jv].astype(jnp.float32), jnp.int32), 21)
                for jv in range(nj)]
            packed = [plsc.load_gather(lut_v, [key[jv]]) for jv in range(nj)]
            thr = [jnp.bitwise_and(packed[jv], 0xFFFFFF) for jv in range(nj)]
            inc = [(d[jv] >= thr[jv]).astype(jnp.int32) for jv in range(nj)]
            bucket = [lax.shift_right_logical(packed[jv], 24) + inc[jv]
                      for jv in range(nj)]
            tw = [plsc.load_gather(tsw_v, [bucket[jv]]) for jv in range(nj)]
            pv = [pos_v[pl.ds(poff + jv * 16, 16)] for jv in range(nj)]
            for jv in range(nj - 1):
                out_bufs[buf][i, pl.ds(jv * 16, 16)] = tw[jv] + pv[jv]
            plsc.store_scatter(out_bufs[buf], [zeros16 + i, tailcols],
                               tw[nj - 1] + pv[nj - 1], mask=tailmask)
            return carry

        lax.fori_loop(0, _NO, row, 0, unroll=8)

    def outer(g, carry):
        for buf in range(2):
            k = g * 2 + buf

            @pl.when(g > 0)
            def _wait():
                pltpu.make_async_copy(
                    out_bufs[buf], out_hbm.at[b0 + k - 2], sems[buf]).wait()

            compute_b(k, buf)
            pltpu.async_copy(out_bufs[buf], out_hbm.at[b0 + k], sems[buf])
        return carry

    lax.fori_loop(0, bpw // 2, outer, 0)
    pltpu.make_async_copy(out_bufs[0], out_hbm.at[b0 + bpw - 2],
                          sems[0]).wait()
    pltpu.make_async_copy(out_bufs[1], out_hbm.at[b0 + bpw - 1],
                          sems[1]).wait()


@jax.jit
def kernel(all_timestamps, ts_w, pos_w):
    B = all_timestamps.shape[0]
    ts_pad = jnp.pad(all_timestamps, ((0, 0), (0, _TS_PAD - _N))).reshape(-1)
    tsw_pad = jnp.pad(ts_w, (0, _TSW_PAD - ts_w.shape[0]))
    pos_pad = jnp.pad(pos_w, (0, _POS_PAD - pos_w.shape[0]))
    lut = jnp.asarray(np.pad(_LUT, (0, _LUT_PAD - _LUT_SIZE)))

    mesh = plsc.VectorSubcoreMesh(core_axis_name="c", subcore_axis_name="s")
    bpw = B // (mesh.num_subcores * mesh.num_cores)

    run = pl.kernel(
        functools.partial(_sc_body, bpw, mesh.num_cores),
        out_type=jax.ShapeDtypeStruct((B, _NO, _NO), jnp.float32),
        mesh=mesh,
        compiler_params=pltpu.CompilerParams(
            needs_layout_passes=False, use_tc_tiling_on_sc=True),
        scratch_types=[
            pltpu.VMEM((bpw * _TS_PAD,), jnp.int32),
            pltpu.VMEM((_TSW_PAD,), jnp.float32),
            pltpu.VMEM((_POS_PAD,), jnp.float32),
            pltpu.VMEM((_LUT_PAD,), jnp.int32),
            pltpu.VMEM((_NO, _NO), jnp.float32),
            pltpu.VMEM((_NO, _NO), jnp.float32),
            pltpu.SemaphoreType.DMA,
            pltpu.SemaphoreType.DMA,
        ],
    )
    return run(ts_pad, tsw_pad, pos_pad, lut)


# speedup vs baseline: 1.0284x; 1.0284x over previous
import functools

import numpy as np
import jax
import jax.numpy as jnp
from jax import lax
from jax.experimental import pallas as pl
from jax.experimental.pallas import tpu as pltpu
from jax.experimental.pallas import tpu_sc as plsc

_N = 200
_NO = _N - 1
_TS_PAD = 208
_POS_PAD = 512
_TSW_PAD = 256
_OUT_WORDS = _NO * _NO
_OUT_STRIDE = 39680
_OUT_BUF = 39680
_LUT_SIZE = 588
_LUT_PAD = 640


def _build_lut() -> np.ndarray:
    d = np.arange(1, 1 << 20, dtype=np.int64)
    f = d.astype(np.float32)
    key = (f.view(np.int32) >> 21).astype(np.int64)
    b = (np.log(f) / np.float32(0.301)).astype(np.int32)
    minb = np.full(_LUT_SIZE, 127, np.int32)
    maxb = np.full(_LUT_SIZE, -1, np.int32)
    np.minimum.at(minb, key, b)
    np.maximum.at(maxb, key, b)
    used = maxb >= 0
    assert np.all((maxb - minb)[used] <= 1), "chunk crosses >1 bucket boundary"
    thr = np.full(_LUT_SIZE, 0xFFFFFF, np.int64)
    grew = b > minb[key]
    np.minimum.at(thr, key[grew], d[grew])
    base = np.where(used, minb, 0).astype(np.int64)
    packed = (base << 24) | thr
    return packed.astype(np.int32)


_LUT = _build_lut()
_THR_BITS = np.pad((_LUT.astype(np.int64) & 0xFFFFFF).astype(np.float32)
                   .view(np.int32), (0, _LUT_PAD - _LUT_SIZE))
_BASE = (_LUT.astype(np.int64) >> 24).astype(np.int32)
_TSW2_IDX = np.zeros(2 * _LUT_SIZE, np.int32)
_TSW2_IDX[0::2] = _BASE
_TSW2_IDX[1::2] = np.minimum(_BASE + 1, 128)
_TSW2_IDX = np.pad(_TSW2_IDX, (0, 1280 - 2 * _LUT_SIZE))


def _sc_body(bpw, nc, ts_hbm, tsw_hbm, pos_hbm, lut_hbm, out_hbm,
             ts_all_v, tsw_v, pos_v, lut_v, out_v0, out_v1, sem0, sem1):
    out_bufs = (out_v0, out_v1)
    wid = lax.axis_index("s") * nc + lax.axis_index("c")
    b0 = wid * bpw
    _sc_work(bpw, b0, ts_hbm, tsw_hbm, pos_hbm, lut_hbm, out_hbm,
             ts_all_v, tsw_v, pos_v, lut_v, out_bufs, (sem0, sem1))


def _sc_work(bpw, b0, ts_hbm, tsw_hbm, pos_hbm, lut_hbm, out_hbm,
             ts_all_v, tsw_v, pos_v, lut_v, out_bufs, sems):
    pltpu.sync_copy(tsw_hbm, tsw_v)
    pltpu.sync_copy(pos_hbm, pos_v)
    pltpu.sync_copy(lut_hbm, lut_v)
    pltpu.sync_copy(ts_hbm.at[pl.ds(b0 * _TS_PAD, bpw * _TS_PAD)], ts_all_v)

    zeros16 = jnp.zeros((16,), jnp.int32)
    ones16 = jnp.full((16,), 1, jnp.int32)
    lane = lax.iota(jnp.int32, 16)
    tailcols = jnp.minimum(lane + 192, _NO - 1)
    tailmask = lane < (_NO - 192)

    nj = 13

    def compute_b(k, buf):
        ts_base = k * _TS_PAD
        tsj = [ts_all_v[pl.ds(ts_base + jv * 16, 16)] for jv in range(nj)]

        def row(i, carry):
            ti = plsc.load_gather(ts_all_v, [zeros16 + (ts_base + i + 1)])
            poff = _NO - i
            x = [ti - tsj[jv] for jv in range(nj)]
            babs = [jnp.bitwise_and(
                plsc.bitcast(x[jv].astype(jnp.float32), jnp.int32),
                0x7FFFFFFF) for jv in range(nj)]
            key = [lax.shift_right_logical(babs[jv], 21) for jv in range(nj)]
            thrb = [plsc.load_gather(lut_v, [key[jv]]) for jv in range(nj)]
            inc = [(babs[jv] >= thrb[jv]).astype(jnp.int32)
                   for jv in range(nj)]
            idx = [key[jv] + key[jv] + inc[jv] for jv in range(nj)]
            tw = [plsc.load_gather(tsw_v, [idx[jv]]) for jv in range(nj)]
            pv = [pos_v[pl.ds(poff + jv * 16, 16)] for jv in range(nj)]
            for jv in range(nj - 1):
                out_bufs[buf][i, pl.ds(jv * 16, 16)] = tw[jv] + pv[jv]
            plsc.store_scatter(out_bufs[buf], [zeros16 + i, tailcols],
                               tw[nj - 1] + pv[nj - 1], mask=tailmask)
            return carry

        lax.fori_loop(0, _NO, row, 0, unroll=4)

    def outer(g, carry):
        for buf in range(2):
            k = g * 2 + buf

            @pl.when(g > 0)
            def _wait():
                pltpu.make_async_copy(
                    out_bufs[buf], out_hbm.at[b0 + k - 2], sems[buf]).wait()

            compute_b(k, buf)
            pltpu.async_copy(out_bufs[buf], out_hbm.at[b0 + k], sems[buf])
        return carry

    lax.fori_loop(0, bpw // 2, outer, 0)
    pltpu.make_async_copy(out_bufs[0], out_hbm.at[b0 + bpw - 2],
                          sems[0]).wait()
    pltpu.make_async_copy(out_bufs[1], out_hbm.at[b0 + bpw - 1],
                          sems[1]).wait()


@jax.jit
def kernel(all_timestamps, ts_w, pos_w):
    B = all_timestamps.shape[0]
    ts_pad = jnp.pad(all_timestamps, ((0, 0), (0, _TS_PAD - _N))).reshape(-1)
    tsw2 = jnp.take(ts_w, jnp.asarray(_TSW2_IDX))
    pos_pad = jnp.pad(pos_w, (0, _POS_PAD - pos_w.shape[0]))
    lut = jnp.asarray(_THR_BITS)

    mesh = plsc.VectorSubcoreMesh(core_axis_name="c", subcore_axis_name="s")
    bpw = B // (mesh.num_subcores * mesh.num_cores)

    run = pl.kernel(
        functools.partial(_sc_body, bpw, mesh.num_cores),
        out_type=jax.ShapeDtypeStruct((B, _NO, _NO), jnp.float32),
        mesh=mesh,
        compiler_params=pltpu.CompilerParams(
            needs_layout_passes=False, use_tc_tiling_on_sc=True),
        scratch_types=[
            pltpu.VMEM((bpw * _TS_PAD,), jnp.int32),
            pltpu.VMEM((1280,), jnp.float32),
            pltpu.VMEM((_POS_PAD,), jnp.float32),
            pltpu.VMEM((_LUT_PAD,), jnp.int32),
            pltpu.VMEM((_NO, _NO), jnp.float32),
            pltpu.VMEM((_NO, _NO), jnp.float32),
            pltpu.SemaphoreType.DMA,
            pltpu.SemaphoreType.DMA,
        ],
    )
    return run(ts_pad, tsw2, pos_pad, lut)
